# Initial kernel scaffold; baseline (speedup 1.0000x reference)
import jax
import jax.numpy as jnp
from jax.experimental import pallas as pl


def kernel(x, edge_index, hyperedge_index, Wl0, bl0, Wr0, Wl1, bl1, Wr1, Wl2, bl2, Wr2, Th0, bh0, Th1, bh1, Th2, bh2, Th3, bh3, Th4, bh4, Wlp, blp):
    raise NotImplementedError("write your pallas kernel here")


# speedup vs baseline: 4.7929x; 4.7929x over previous
import functools

import jax
import jax.numpy as jnp
from jax import lax
from jax.experimental import pallas as pl
from jax.experimental.pallas import tpu as pltpu
from jax.experimental.pallas import tpu_sc as plsc

NS = 16
NC = 2
L = 16

N = 10000
E = 320000
NNZ = 160000
B = 400

_SC_PARAMS = pltpu.CompilerParams(use_tc_tiling_on_sc=False)


def _mesh():
  return plsc.VectorSubcoreMesh(core_axis_name="c", subcore_axis_name="s")


def _tile_ranges(n_out):
  main = (-(-n_out // NS) + 7) // 8 * 8
  last = n_out - main * (NS - 1)
  assert 0 < last <= main
  return main, last


@functools.lru_cache(maxsize=None)
def _make_segsum(nslots, w):
  main, last = _tile_ranges(N)

  @functools.partial(
      pl.kernel,
      out_type=jax.ShapeDtypeStruct((NC, N, w), jnp.float32),
      mesh=_mesh(),
      compiler_params=_SC_PARAMS,
      scratch_types=[
          pltpu.VMEM((L,), jnp.int32),
          pltpu.VMEM((B,), jnp.int32),
          pltpu.VMEM((B,), jnp.int32),
          pltpu.VMEM((B, w), jnp.float32),
          pltpu.VMEM_SHARED((N, w), jnp.float32),
          pltpu.SemaphoreType.DMA,
      ],
  )
  def k(tab_hbm, src_hbm, dst_hbm, par_hbm, out_hbm,
        par_v, src_v, dst_v, rows_v, acc, sem):
    c = lax.axis_index("c")
    s = lax.axis_index("s")
    pltpu.sync_copy(par_hbm, par_v)
    pv = par_v[pl.ds(0, L)]
    nch = pv[0]
    slot_base = pv[1]

    zv = jnp.zeros((L,), jnp.float32)

    def zrow(i, carry):
      for t in range(w // L):
        rows_v[i, pl.ds(t * L, L)] = zv
      return carry

    lax.fori_loop(0, B, zrow, 0)

    base_r = pl.multiple_of(s * main, 8)

    @pl.when(s < NS - 1)
    def _():
      pltpu.sync_copy(rows_v.at[pl.ds(0, B)], acc.at[pl.ds(base_r, B)])
      pltpu.sync_copy(rows_v.at[pl.ds(0, main - B)],
                      acc.at[pl.ds(base_r + B, main - B)])

    @pl.when(s == NS - 1)
    def _():
      pltpu.sync_copy(rows_v.at[pl.ds(0, B)], acc.at[pl.ds(base_r, B)])
      pltpu.sync_copy(rows_v.at[pl.ds(0, last - B)],
                      acc.at[pl.ds(base_r + B, last - B)])

    plsc.subcore_barrier()

    row_off = (slot_base + c) * N

    def body(j, carry):
      e0 = pl.multiple_of(s * nch * B + j * B, 8)
      pltpu.sync_copy(src_hbm.at[pl.ds(e0, B)], src_v)
      pltpu.sync_copy(dst_hbm.at[pl.ds(e0, B)], dst_v)
      for t in range(B // L):
        sl = pl.ds(t * L, L)
        src_v[sl] = src_v[sl] + row_off
      pltpu.async_copy(tab_hbm.at[src_v], rows_v, sem).wait()
      pltpu.sync_copy(rows_v, acc.at[dst_v], add=True)
      return carry

    lax.fori_loop(0, nch, body, 0)
    plsc.subcore_barrier()

    @pl.when(s < NS - 1)
    def _():
      pltpu.sync_copy(acc.at[pl.ds(base_r, main)],
                      out_hbm.at[c, pl.ds(base_r, main)])

    @pl.when(s == NS - 1)
    def _():
      pltpu.sync_copy(acc.at[pl.ds(base_r, last)],
                      out_hbm.at[c, pl.ds(base_r, last)])

  return k


def _segsum(tab, src, dst, nch, slot_base):
  nslots, n, w = tab.shape
  assert n == N
  par = jnp.zeros((L,), jnp.int32).at[0].set(nch).at[1].set(slot_base)
  k = _make_segsum(nslots, w)
  return k(tab.reshape(nslots * N, w), src, dst, par)


@functools.lru_cache(maxsize=None)
def _make_counts():
  Bc = 1000
  ept_e = E // (NC * NS)
  ept_z = NNZ // (NC * NS)
  nch_e = ept_e // Bc
  nch_z = ept_z // Bc
  main, last = _tile_ranges(N)
  assert main <= Bc

  otype = [
      jax.ShapeDtypeStruct((NC, N, L), jnp.float32),
      jax.ShapeDtypeStruct((NC, N, L), jnp.float32),
      jax.ShapeDtypeStruct((NC, N, L), jnp.float32),
  ]

  @functools.partial(
      pl.kernel,
      out_type=otype,
      mesh=_mesh(),
      compiler_params=_SC_PARAMS,
      scratch_types=[
          pltpu.VMEM((Bc,), jnp.int32),
          pltpu.VMEM((Bc, L), jnp.float32),
          pltpu.VMEM_SHARED((N, L), jnp.float32),
          pltpu.VMEM_SHARED((N, L), jnp.float32),
          pltpu.VMEM_SHARED((N, L), jnp.float32),
      ],
  )
  def k(dst_hbm, node_hbm, he_hbm, cnt_out, dg_out, bg_out,
        idx_v, ones_v, acc_c, acc_d, acc_b):
    c = lax.axis_index("c")
    s = lax.axis_index("s")
    wid = c * NS + s

    zv = jnp.zeros((L,), jnp.float32)

    def zrow(i, carry):
      ones_v[i] = zv
      return carry

    lax.fori_loop(0, Bc, zrow, 0)

    base_r = pl.multiple_of(s * main, 8)

    @pl.when(s < NS - 1)
    def _():
      for a in (acc_c, acc_d, acc_b):
        pltpu.sync_copy(ones_v.at[pl.ds(0, main)], a.at[pl.ds(base_r, main)])

    @pl.when(s == NS - 1)
    def _():
      for a in (acc_c, acc_d, acc_b):
        pltpu.sync_copy(ones_v.at[pl.ds(0, last)], a.at[pl.ds(base_r, last)])

    ov = jnp.ones((L,), jnp.float32)

    def orow(i, carry):
      ones_v[i] = ov
      return carry

    lax.fori_loop(0, Bc, orow, 0)
    plsc.subcore_barrier()

    def mk_body(src_ref, acc, ept):
      def body(j, carry):
        e0 = pl.multiple_of(wid * ept + j * Bc, 8)
        pltpu.sync_copy(src_ref.at[pl.ds(e0, Bc)], idx_v)
        pltpu.sync_copy(ones_v, acc.at[idx_v], add=True)
        return carry
      return body

    lax.fori_loop(0, nch_e, mk_body(dst_hbm, acc_c, ept_e), 0)
    lax.fori_loop(0, nch_z, mk_body(node_hbm, acc_d, ept_z), 0)
    lax.fori_loop(0, nch_z, mk_body(he_hbm, acc_b, ept_z), 0)
    plsc.subcore_barrier()

    @pl.when(s < NS - 1)
    def _():
      for a, o in ((acc_c, cnt_out), (acc_d, dg_out), (acc_b, bg_out)):
        pltpu.sync_copy(a.at[pl.ds(base_r, main)],
                        o.at[c, pl.ds(base_r, main)])

    @pl.when(s == NS - 1)
    def _():
      for a, o in ((acc_c, cnt_out), (acc_d, dg_out), (acc_b, bg_out)):
        pltpu.sync_copy(a.at[pl.ds(base_r, last)],
                        o.at[c, pl.ds(base_r, last)])

  return k


_RB = 1000


def _scale_from(counts_blk, kind):
  tot = counts_blk[0] + counts_blk[1]
  col = tot[:, 0:1]
  if kind == "mean":
    return 1.0 / jnp.maximum(col, 1.0)
  return jnp.where(col > 0, 1.0 / jnp.maximum(col, 1e-30), 0.0)


def _tc_block(aggs, scale2, kind, xins, w_a, w_x, bias, relu, out_w):
  na = len(aggs)
  xslots = [ns for (_, ns) in xins]
  nx = len(xins)
  grid = N // _RB

  def body(*refs):
    agg_refs = refs[:na]
    pos = na
    if na:
      scale_ref = refs[pos]; pos += 1
    x_refs = refs[pos:pos + nx]; pos += nx
    wa_ref = None
    if na:
      wa_ref = refs[pos]; pos += 1
    wx_ref = None
    if nx:
      wx_ref = refs[pos]; pos += 1
    b_ref = refs[pos]
    out_ref = refs[pos + 1]

    acc = None
    if na:
      sv = _scale_from(scale_ref, kind)
      for p in range(na):
        for h in range(2):
          t = jnp.dot(agg_refs[p][h] * sv, wa_ref[2 * p + h],
                      preferred_element_type=jnp.float32)
          acc = t if acc is None else acc + t
    ko = 0
    for q in range(nx):
      for h in range(xslots[q]):
        t = jnp.dot(x_refs[q][h], wx_ref[ko],
                    preferred_element_type=jnp.float32)
        ko += 1
        acc = t if acc is None else acc + t
    res = acc + b_ref[0][None, :]
    if relu:
      res = jnp.maximum(res, 0.0)
    if out_w == 256:
      out_ref[0] = res[:, 0:64]
      out_ref[1] = res[:, 128:192]
      out_ref[2] = res[:, 64:128]
      out_ref[3] = res[:, 192:256]
    else:
      h = out_w // 2
      out_ref[0] = res[:, :h]
      out_ref[1] = res[:, h:]

  in_specs = [pl.BlockSpec((2, _RB, 64), lambda i: (0, i, 0)) for _ in aggs]
  args = list(aggs)
  if na:
    in_specs.append(pl.BlockSpec((2, _RB, L), lambda i: (0, i, 0)))
    args.append(scale2)
  for (arr, ns) in xins:
    in_specs.append(
        pl.BlockSpec((ns, _RB, 64), lambda i: (0, i, 0)))
    args.append(arr)
  if na:
    in_specs.append(pl.BlockSpec(w_a.shape, lambda i: (0, 0, 0)))
    args.append(w_a)
  if nx:
    in_specs.append(pl.BlockSpec(w_x.shape, lambda i: (0, 0, 0)))
    args.append(w_x)
  in_specs.append(pl.BlockSpec((1, out_w), lambda i: (0, 0)))
  args.append(bias.reshape(1, out_w))

  if out_w == 256:
    oshape, oblk = (4, N, 64), (4, _RB, 64)
  else:
    oshape, oblk = (2, N, out_w // 2), (2, _RB, out_w // 2)

  return pl.pallas_call(
      body,
      grid=(grid,),
      in_specs=in_specs,
      out_specs=pl.BlockSpec(oblk, lambda i: (0, i, 0)),
      out_shape=jax.ShapeDtypeStruct(oshape, jnp.float32),
  )(*args)


def _tc_escale(parts, bg2, w, out_slots):
  nparts = len(parts)
  grid = N // _RB

  def body(*refs):
    e_refs = refs[:nparts]
    bg_ref = refs[nparts]
    out_ref = refs[nparts + 1]
    bi = _scale_from(bg_ref, "inv")
    k = 0
    for p in range(nparts):
      for h in range(2):
        out_ref[k] = e_refs[p][h] * bi
        k += 1
    while k < out_slots:
      out_ref[k] = jnp.zeros((_RB, w), jnp.float32)
      k += 1

  in_specs = [pl.BlockSpec((2, _RB, w), lambda i: (0, i, 0))
              for _ in parts]
  in_specs.append(pl.BlockSpec((2, _RB, L), lambda i: (0, i, 0)))
  return pl.pallas_call(
      body,
      grid=(grid,),
      in_specs=in_specs,
      out_specs=pl.BlockSpec((out_slots, _RB, w), lambda i: (0, i, 0)),
      out_shape=jax.ShapeDtypeStruct((out_slots, N, w), jnp.float32),
  )(*parts, bg2)


def _tc_final(a3, cnt2, rs, m5, dg2, bh4p, wsp, whp, blpp, n_cls):
  grid = N // _RB

  def body(a_ref, cnt_ref, rs_ref, m_ref, dg_ref, bh_ref, ws_ref, wh_ref,
           bp_ref, out_ref):
    ci = _scale_from(cnt_ref, "mean")
    di = _scale_from(dg_ref, "inv")
    acat = jnp.concatenate([a_ref[0], a_ref[1]], axis=1)
    rcat = jnp.concatenate([rs_ref[0], rs_ref[1]], axis=1)
    mcat = jnp.concatenate([m_ref[0], m_ref[1]], axis=1)
    xs = acat * ci + rcat
    xh = mcat * di + bh_ref[0][None, :]
    z = (
        jnp.dot(xs, ws_ref[...], preferred_element_type=jnp.float32)
        + jnp.dot(xh, wh_ref[...], preferred_element_type=jnp.float32)
        + bp_ref[0][None, :]
    )
    mask = lax.broadcasted_iota(jnp.int32, z.shape, 1) < n_cls
    zm = jnp.where(mask, z, -jnp.inf)
    mx = jnp.max(zm, axis=1, keepdims=True)
    se = jnp.sum(jnp.where(mask, jnp.exp(z - mx), 0.0), axis=1, keepdims=True)
    out_ref[...] = z - mx - jnp.log(se)

  return pl.pallas_call(
      body,
      grid=(grid,),
      in_specs=[
          pl.BlockSpec((2, _RB, 32), lambda i: (0, i, 0)),
          pl.BlockSpec((2, _RB, L), lambda i: (0, i, 0)),
          pl.BlockSpec((2, _RB, 32), lambda i: (0, i, 0)),
          pl.BlockSpec((2, _RB, 32), lambda i: (0, i, 0)),
          pl.BlockSpec((2, _RB, L), lambda i: (0, i, 0)),
          pl.BlockSpec((1, 64), lambda i: (0, 0)),
          pl.BlockSpec((64, 128), lambda i: (0, 0)),
          pl.BlockSpec((64, 128), lambda i: (0, 0)),
          pl.BlockSpec((1, 128), lambda i: (0, 0)),
      ],
      out_specs=pl.BlockSpec((_RB, 128), lambda i: (i, 0)),
      out_shape=jax.ShapeDtypeStruct((N, 128), jnp.float32),
  )(a3, cnt2, rs, m5, dg2, bh4p, wsp, whp, blpp)


def _qrows(w, qs):
  return jnp.stack([w[64 * q:64 * (q + 1)] for q in qs], axis=0)


def _pad_cols(a, w):
  return jnp.pad(a, ((0, 0), (0, w - a.shape[1])))


_PERM = (0, 2, 1, 3)


@jax.jit
def kernel(x, edge_index, hyperedge_index, Wl0, bl0, Wr0, Wl1, bl1, Wr1,
           Wl2, bl2, Wr2, Th0, bh0, Th1, bh1, Th2, bh2, Th3, bh3, Th4, bh4,
           Wlp, blp):
  c_cls = Wl2.shape[1]

  src, dst = edge_index[0], edge_index[1]
  node, he = hyperedge_index[0], hyperedge_index[1]
  node_p = jnp.pad(node, (0, E - NNZ))
  he_p = jnp.pad(he, (0, E - NNZ))
  nch_e = E // (NS * B)
  nch_z = NNZ // (NS * B)

  cnt2, dg2, bg2 = _make_counts()(dst, node, he)

  xs = jnp.stack([x[:, :64], x[:, 64:]], axis=0)
  xs4 = jnp.concatenate([xs, jnp.zeros((2, N, 64), jnp.float32)], axis=0)

  a1 = _segsum(xs4, src, dst, nch_e, 0)
  h1 = _tc_block([a1], cnt2, "mean", [(xs4, 2)], _qrows(Wl0, (0, 1)),
                 _qrows(Wr0, (0, 1)), bl0, True, 256)
  a2A = _segsum(h1, src, dst, nch_e, 0)
  a2B = _segsum(h1, src, dst, nch_e, 2)
  h2 = _tc_block([a2A, a2B], cnt2, "mean", [(h1, 4)], _qrows(Wl1, _PERM),
                 _qrows(Wr1, _PERM), bl1, True, 256)
  p = _tc_block([], None, None, [(h2, 4)], None,
                _qrows(_pad_cols(Wl2, 64), _PERM),
                jnp.zeros((64,), jnp.float32), False, 64)
  rs = _tc_block([], None, None, [(h2, 4)], None,
                 _qrows(_pad_cols(Wr2, 64), _PERM),
                 _pad_cols(bl2.reshape(1, -1), 64)[0], False, 64)
  a3 = _segsum(p, src, dst, nch_e, 0)

  g4 = xs4
  g_slots = 2
  for li, (th, bh) in enumerate(((Th0, bh0), (Th1, bh1), (Th2, bh2),
                                 (Th3, bh3))):
    if g_slots == 2:
      eA = _segsum(g4, node_p, he_p, nch_z, 0)
      ep = _tc_escale([eA], bg2, 64, 4)
      m = _segsum(ep, he_p, node_p, nch_z, 0)
      g4 = _tc_block([m], dg2, "inv", [], _qrows(th, (0, 1)), None,
                     bh, True, 256)
    else:
      eA = _segsum(g4, node_p, he_p, nch_z, 0)
      eB = _segsum(g4, node_p, he_p, nch_z, 2)
      ep = _tc_escale([eA, eB], bg2, 64, 4)
      mA = _segsum(ep, he_p, node_p, nch_z, 0)
      mB = _segsum(ep, he_p, node_p, nch_z, 2)
      g4 = _tc_block([mA, mB], dg2, "inv", [], _qrows(th, _PERM), None,
                     bh, True, 256)
    g_slots = 4
  p4 = _tc_block([], None, None, [(g4, 4)], None,
                 _qrows(_pad_cols(Th4, 64), _PERM),
                 jnp.zeros((64,), jnp.float32), False, 64)
  e5 = _segsum(p4, node_p, he_p, nch_z, 0)
  e5p = _tc_escale([e5], bg2, 32, 2)
  m5 = _segsum(e5p, he_p, node_p, nch_z, 0)

  wsp = jnp.zeros((64, 128), jnp.float32).at[:c_cls, :c_cls].set(Wlp[:c_cls])
  whp = jnp.zeros((64, 128), jnp.float32).at[:c_cls, :c_cls].set(Wlp[c_cls:])
  blpp = jnp.zeros((1, 128), jnp.float32).at[0, :c_cls].set(blp)
  bh4p = _pad_cols(bh4.reshape(1, -1), 64)
  out128 = _tc_final(a3, cnt2, rs, m5, dg2, bh4p, wsp, whp, blpp, c_cls)
  return out128[:, :c_cls]


# speedup vs baseline: 7.6416x; 1.5944x over previous
import functools

import jax
import jax.numpy as jnp
from jax import lax
from jax.experimental import pallas as pl
from jax.experimental.pallas import tpu as pltpu
from jax.experimental.pallas import tpu_sc as plsc

NS = 16
NC = 2
L = 16

N = 10000
E = 320000
NNZ = 160000
B = 400

_SC_PARAMS = pltpu.CompilerParams(use_tc_tiling_on_sc=False)


def _mesh():
  return plsc.VectorSubcoreMesh(core_axis_name="c", subcore_axis_name="s")


def _tile_ranges(n_out):
  main = (-(-n_out // NS) + 7) // 8 * 8
  last = n_out - main * (NS - 1)
  assert 0 < last <= main
  return main, last


_NCH_MAX = E // (NS * B)


@functools.lru_cache(maxsize=None)
def _make_segsum(nslots, w):
  main, last = _tile_ranges(N)

  @functools.partial(
      pl.kernel,
      out_type=jax.ShapeDtypeStruct((NC, N, w), jnp.float32),
      mesh=_mesh(),
      compiler_params=_SC_PARAMS,
      scratch_types=[
          pltpu.VMEM((L,), jnp.int32),
          pltpu.VMEM((_NCH_MAX, B), jnp.int32),
          pltpu.VMEM((2, B), jnp.int32),
          pltpu.VMEM((2, B, w), jnp.float32),
          pltpu.VMEM_SHARED((N, w), jnp.float32),
          pltpu.SemaphoreType.DMA,
          pltpu.SemaphoreType.DMA,
          pltpu.SemaphoreType.DMA,
          pltpu.SemaphoreType.DMA,
      ],
  )
  def k(tab_hbm, src_hbm, dst_hbm, par_hbm, out_hbm,
        par_v, src_v, dst_v, rows_v, acc, sem0, sem1, semd0, semd1):
    c = lax.axis_index("c")
    s = lax.axis_index("s")
    pltpu.sync_copy(par_hbm, par_v)
    pv = par_v[pl.ds(0, L)]
    nch = pv[0]
    slot_base = pv[1]

    zv = jnp.zeros((L,), jnp.float32)

    def zrow(i, carry):
      for t in range(w // L):
        rows_v[0, i, pl.ds(t * L, L)] = zv
      return carry

    lax.fori_loop(0, B, zrow, 0)

    base_r = pl.multiple_of(s * main, 8)

    @pl.when(s < NS - 1)
    def _():
      pltpu.sync_copy(rows_v.at[0, pl.ds(0, B)], acc.at[pl.ds(base_r, B)])
      pltpu.sync_copy(rows_v.at[0, pl.ds(0, main - B)],
                      acc.at[pl.ds(base_r + B, main - B)])

    @pl.when(s == NS - 1)
    def _():
      pltpu.sync_copy(rows_v.at[0, pl.ds(0, B)], acc.at[pl.ds(base_r, B)])
      pltpu.sync_copy(rows_v.at[0, pl.ds(0, last - B)],
                      acc.at[pl.ds(base_r + B, last - B)])

    pltpu.sync_copy(src_hbm.at[pl.ds(s * nch, _NCH_MAX)], src_v)

    row_off = (slot_base + c) * N

    def offrow(r, carry):
      for t in range(B // L):
        sl = pl.ds(t * L, L)
        src_v[r, sl] = src_v[r, sl] + row_off
      return carry

    lax.fori_loop(0, _NCH_MAX, offrow, 0)
    plsc.subcore_barrier()

    e_base = pl.multiple_of(s * nch * B, 8)

    def fetch(j, buf, semg, semd):
      pltpu.async_copy(dst_hbm.at[pl.ds(e_base + j * B, B)],
                       dst_v.at[buf], semd)
      pltpu.async_copy(tab_hbm.at[src_v.at[j]], rows_v.at[buf], semg)

    def drain(j, buf, semg, semd):
      pltpu.make_async_copy(dst_hbm.at[pl.ds(e_base, B)],
                            dst_v.at[buf], semd).wait()
      pltpu.make_async_copy(tab_hbm.at[src_v.at[0]],
                            rows_v.at[buf], semg).wait()
      pltpu.sync_copy(rows_v.at[buf], acc.at[dst_v.at[buf]], add=True)

    @pl.when(nch > 0)
    def _():
      fetch(0, 0, sem0, semd0)

    def body(j, carry):
      even = j % 2 == 0
      more = j + 1 < nch

      @pl.when(more & even)
      def _():
        fetch(j + 1, 1, sem1, semd1)

      @pl.when(more & (~even))
      def _():
        fetch(j + 1, 0, sem0, semd0)

      @pl.when(even)
      def _():
        drain(j, 0, sem0, semd0)

      @pl.when(~even)
      def _():
        drain(j, 1, sem1, semd1)

      return carry

    lax.fori_loop(0, nch, body, 0)
    plsc.subcore_barrier()

    @pl.when(s < NS - 1)
    def _():
      pltpu.sync_copy(acc.at[pl.ds(base_r, main)],
                      out_hbm.at[c, pl.ds(base_r, main)])

    @pl.when(s == NS - 1)
    def _():
      pltpu.sync_copy(acc.at[pl.ds(base_r, last)],
                      out_hbm.at[c, pl.ds(base_r, last)])

  return k


def _segsum(tab, src2d, dst2d, nch, slot_base):
  nslots, n, w = tab.shape
  assert n == N
  par = jnp.zeros((L,), jnp.int32).at[0].set(nch).at[1].set(slot_base)
  k = _make_segsum(nslots, w)
  return k(tab.reshape(nslots * N, w), src2d, dst2d, par)


@functools.lru_cache(maxsize=None)
def _make_counts():
  Bc = 1000
  ept_e = E // (NC * NS)
  ept_z = NNZ // (NC * NS)
  nch_e = ept_e // Bc
  nch_z = ept_z // Bc
  main, last = _tile_ranges(N)
  assert main <= Bc

  otype = [
      jax.ShapeDtypeStruct((NC, N, L), jnp.float32),
      jax.ShapeDtypeStruct((NC, N, L), jnp.float32),
      jax.ShapeDtypeStruct((NC, N, L), jnp.float32),
  ]

  @functools.partial(
      pl.kernel,
      out_type=otype,
      mesh=_mesh(),
      compiler_params=_SC_PARAMS,
      scratch_types=[
          pltpu.VMEM((Bc,), jnp.int32),
          pltpu.VMEM((Bc, L), jnp.float32),
          pltpu.VMEM_SHARED((N, L), jnp.float32),
          pltpu.VMEM_SHARED((N, L), jnp.float32),
          pltpu.VMEM_SHARED((N, L), jnp.float32),
      ],
  )
  def k(dst_hbm, node_hbm, he_hbm, cnt_out, dg_out, bg_out,
        idx_v, ones_v, acc_c, acc_d, acc_b):
    c = lax.axis_index("c")
    s = lax.axis_index("s")
    wid = c * NS + s

    zv = jnp.zeros((L,), jnp.float32)

    def zrow(i, carry):
      ones_v[i] = zv
      return carry

    lax.fori_loop(0, Bc, zrow, 0)

    base_r = pl.multiple_of(s * main, 8)

    @pl.when(s < NS - 1)
    def _():
      for a in (acc_c, acc_d, acc_b):
        pltpu.sync_copy(ones_v.at[pl.ds(0, main)], a.at[pl.ds(base_r, main)])

    @pl.when(s == NS - 1)
    def _():
      for a in (acc_c, acc_d, acc_b):
        pltpu.sync_copy(ones_v.at[pl.ds(0, last)], a.at[pl.ds(base_r, last)])

    ov = jnp.ones((L,), jnp.float32)

    def orow(i, carry):
      ones_v[i] = ov
      return carry

    lax.fori_loop(0, Bc, orow, 0)
    plsc.subcore_barrier()

    def mk_body(src_ref, acc, ept):
      def body(j, carry):
        e0 = pl.multiple_of(wid * ept + j * Bc, 8)
        pltpu.sync_copy(src_ref.at[pl.ds(e0, Bc)], idx_v)
        pltpu.sync_copy(ones_v, acc.at[idx_v], add=True)
        return carry
      return body

    lax.fori_loop(0, nch_e, mk_body(dst_hbm, acc_c, ept_e), 0)
    lax.fori_loop(0, nch_z, mk_body(node_hbm, acc_d, ept_z), 0)
    lax.fori_loop(0, nch_z, mk_body(he_hbm, acc_b, ept_z), 0)
    plsc.subcore_barrier()

    @pl.when(s < NS - 1)
    def _():
      for a, o in ((acc_c, cnt_out), (acc_d, dg_out), (acc_b, bg_out)):
        pltpu.sync_copy(a.at[pl.ds(base_r, main)],
                        o.at[c, pl.ds(base_r, main)])

    @pl.when(s == NS - 1)
    def _():
      for a, o in ((acc_c, cnt_out), (acc_d, dg_out), (acc_b, bg_out)):
        pltpu.sync_copy(a.at[pl.ds(base_r, last)],
                        o.at[c, pl.ds(base_r, last)])

  return k


_RB = 1000


def _scale_from(counts_blk, kind):
  tot = counts_blk[0] + counts_blk[1]
  col = tot[:, 0:1]
  if kind == "mean":
    return 1.0 / jnp.maximum(col, 1.0)
  return jnp.where(col > 0, 1.0 / jnp.maximum(col, 1e-30), 0.0)


def _tc_block(aggs, scale2, kind, xins, w_a, w_x, bias, relu, out_w):
  na = len(aggs)
  xslots = [ns for (_, ns) in xins]
  nx = len(xins)
  grid = N // _RB

  def body(*refs):
    agg_refs = refs[:na]
    pos = na
    if na:
      scale_ref = refs[pos]; pos += 1
    x_refs = refs[pos:pos + nx]; pos += nx
    wa_ref = None
    if na:
      wa_ref = refs[pos]; pos += 1
    wx_ref = None
    if nx:
      wx_ref = refs[pos]; pos += 1
    b_ref = refs[pos]
    out_ref = refs[pos + 1]

    acc = None
    if na:
      sv = _scale_from(scale_ref, kind)
      for p in range(na):
        for h in range(2):
          t = jnp.dot(agg_refs[p][h] * sv, wa_ref[2 * p + h],
                      preferred_element_type=jnp.float32)
          acc = t if acc is None else acc + t
    ko = 0
    for q in range(nx):
      for h in range(xslots[q]):
        t = jnp.dot(x_refs[q][h], wx_ref[ko],
                    preferred_element_type=jnp.float32)
        ko += 1
        acc = t if acc is None else acc + t
    res = acc + b_ref[0][None, :]
    if relu:
      res = jnp.maximum(res, 0.0)
    if out_w == 256:
      out_ref[0] = res[:, 0:64]
      out_ref[1] = res[:, 128:192]
      out_ref[2] = res[:, 64:128]
      out_ref[3] = res[:, 192:256]
    else:
      h = out_w // 2
      out_ref[0] = res[:, :h]
      out_ref[1] = res[:, h:]

  in_specs = [pl.BlockSpec((2, _RB, 64), lambda i: (0, i, 0)) for _ in aggs]
  args = list(aggs)
  if na:
    in_specs.append(pl.BlockSpec((2, _RB, L), lambda i: (0, i, 0)))
    args.append(scale2)
  for (arr, ns) in xins:
    in_specs.append(
        pl.BlockSpec((ns, _RB, 64), lambda i: (0, i, 0)))
    args.append(arr)
  if na:
    in_specs.append(pl.BlockSpec(w_a.shape, lambda i: (0, 0, 0)))
    args.append(w_a)
  if nx:
    in_specs.append(pl.BlockSpec(w_x.shape, lambda i: (0, 0, 0)))
    args.append(w_x)
  in_specs.append(pl.BlockSpec((1, out_w), lambda i: (0, 0)))
  args.append(bias.reshape(1, out_w))

  if out_w == 256:
    oshape, oblk = (4, N, 64), (4, _RB, 64)
  else:
    oshape, oblk = (2, N, out_w // 2), (2, _RB, out_w // 2)

  return pl.pallas_call(
      body,
      grid=(grid,),
      in_specs=in_specs,
      out_specs=pl.BlockSpec(oblk, lambda i: (0, i, 0)),
      out_shape=jax.ShapeDtypeStruct(oshape, jnp.float32),
  )(*args)


def _tc_escale(parts, bg2, w, out_slots):
  nparts = len(parts)
  grid = N // _RB

  def body(*refs):
    e_refs = refs[:nparts]
    bg_ref = refs[nparts]
    out_ref = refs[nparts + 1]
    bi = _scale_from(bg_ref, "inv")
    k = 0
    for p in range(nparts):
      for h in range(2):
        out_ref[k] = e_refs[p][h] * bi
        k += 1
    while k < out_slots:
      out_ref[k] = jnp.zeros((_RB, w), jnp.float32)
      k += 1

  in_specs = [pl.BlockSpec((2, _RB, w), lambda i: (0, i, 0))
              for _ in parts]
  in_specs.append(pl.BlockSpec((2, _RB, L), lambda i: (0, i, 0)))
  return pl.pallas_call(
      body,
      grid=(grid,),
      in_specs=in_specs,
      out_specs=pl.BlockSpec((out_slots, _RB, w), lambda i: (0, i, 0)),
      out_shape=jax.ShapeDtypeStruct((out_slots, N, w), jnp.float32),
  )(*parts, bg2)


def _tc_final(a3, cnt2, rs, m5, dg2, bh4p, wsp, whp, blpp, n_cls):
  grid = N // _RB

  def body(a_ref, cnt_ref, rs_ref, m_ref, dg_ref, bh_ref, ws_ref, wh_ref,
           bp_ref, out_ref):
    ci = _scale_from(cnt_ref, "mean")
    di = _scale_from(dg_ref, "inv")
    acat = jnp.concatenate([a_ref[0], a_ref[1]], axis=1)
    rcat = jnp.concatenate([rs_ref[0], rs_ref[1]], axis=1)
    mcat = jnp.concatenate([m_ref[0], m_ref[1]], axis=1)
    xs = acat * ci + rcat
    xh = mcat * di + bh_ref[0][None, :]
    z = (
        jnp.dot(xs, ws_ref[...], preferred_element_type=jnp.float32)
        + jnp.dot(xh, wh_ref[...], preferred_element_type=jnp.float32)
        + bp_ref[0][None, :]
    )
    mask = lax.broadcasted_iota(jnp.int32, z.shape, 1) < n_cls
    zm = jnp.where(mask, z, -jnp.inf)
    mx = jnp.max(zm, axis=1, keepdims=True)
    se = jnp.sum(jnp.where(mask, jnp.exp(z - mx), 0.0), axis=1, keepdims=True)
    out_ref[...] = z - mx - jnp.log(se)

  return pl.pallas_call(
      body,
      grid=(grid,),
      in_specs=[
          pl.BlockSpec((2, _RB, 32), lambda i: (0, i, 0)),
          pl.BlockSpec((2, _RB, L), lambda i: (0, i, 0)),
          pl.BlockSpec((2, _RB, 32), lambda i: (0, i, 0)),
          pl.BlockSpec((2, _RB, 32), lambda i: (0, i, 0)),
          pl.BlockSpec((2, _RB, L), lambda i: (0, i, 0)),
          pl.BlockSpec((1, 64), lambda i: (0, 0)),
          pl.BlockSpec((64, 128), lambda i: (0, 0)),
          pl.BlockSpec((64, 128), lambda i: (0, 0)),
          pl.BlockSpec((1, 128), lambda i: (0, 0)),
      ],
      out_specs=pl.BlockSpec((_RB, 128), lambda i: (i, 0)),
      out_shape=jax.ShapeDtypeStruct((N, 128), jnp.float32),
  )(a3, cnt2, rs, m5, dg2, bh4p, wsp, whp, blpp)


def _qrows(w, qs):
  return jnp.stack([w[64 * q:64 * (q + 1)] for q in qs], axis=0)


def _pad_cols(a, w):
  return jnp.pad(a, ((0, 0), (0, w - a.shape[1])))


_PERM = (0, 2, 1, 3)


@jax.jit
def kernel(x, edge_index, hyperedge_index, Wl0, bl0, Wr0, Wl1, bl1, Wr1,
           Wl2, bl2, Wr2, Th0, bh0, Th1, bh1, Th2, bh2, Th3, bh3, Th4, bh4,
           Wlp, blp):
  c_cls = Wl2.shape[1]

  src, dst = edge_index[0], edge_index[1]
  node, he = hyperedge_index[0], hyperedge_index[1]
  shp = (NS * _NCH_MAX, B)
  src2 = src.reshape(shp)
  node1 = jnp.pad(node, (0, E - NNZ))
  he1 = jnp.pad(he, (0, E - NNZ))
  node2 = node1.reshape(shp)
  he2 = he1.reshape(shp)
  nch_e = E // (NS * B)
  nch_z = NNZ // (NS * B)

  cnt2, dg2, bg2 = _make_counts()(dst, node, he)

  xs = jnp.stack([x[:, :64], x[:, 64:]], axis=0)
  xs4 = jnp.concatenate([xs, jnp.zeros((2, N, 64), jnp.float32)], axis=0)

  a1 = _segsum(xs4, src2, dst, nch_e, 0)
  h1 = _tc_block([a1], cnt2, "mean", [(xs4, 2)], _qrows(Wl0, (0, 1)),
                 _qrows(Wr0, (0, 1)), bl0, True, 256)
  a2A = _segsum(h1, src2, dst, nch_e, 0)
  a2B = _segsum(h1, src2, dst, nch_e, 2)
  h2 = _tc_block([a2A, a2B], cnt2, "mean", [(h1, 4)], _qrows(Wl1, _PERM),
                 _qrows(Wr1, _PERM), bl1, True, 256)
  p = _tc_block([], None, None, [(h2, 4)], None,
                _qrows(_pad_cols(Wl2, 64), _PERM),
                jnp.zeros((64,), jnp.float32), False, 64)
  rs = _tc_block([], None, None, [(h2, 4)], None,
                 _qrows(_pad_cols(Wr2, 64), _PERM),
                 _pad_cols(bl2.reshape(1, -1), 64)[0], False, 64)
  a3 = _segsum(p, src2, dst, nch_e, 0)

  g4 = xs4
  g_slots = 2
  for li, (th, bh) in enumerate(((Th0, bh0), (Th1, bh1), (Th2, bh2),
                                 (Th3, bh3))):
    if g_slots == 2:
      eA = _segsum(g4, node2, he1, nch_z, 0)
      ep = _tc_escale([eA], bg2, 64, 4)
      m = _segsum(ep, he2, node1, nch_z, 0)
      g4 = _tc_block([m], dg2, "inv", [], _qrows(th, (0, 1)), None,
                     bh, True, 256)
    else:
      eA = _segsum(g4, node2, he1, nch_z, 0)
      eB = _segsum(g4, node2, he1, nch_z, 2)
      ep = _tc_escale([eA, eB], bg2, 64, 4)
      mA = _segsum(ep, he2, node1, nch_z, 0)
      mB = _segsum(ep, he2, node1, nch_z, 2)
      g4 = _tc_block([mA, mB], dg2, "inv", [], _qrows(th, _PERM), None,
                     bh, True, 256)
    g_slots = 4
  p4 = _tc_block([], None, None, [(g4, 4)], None,
                 _qrows(_pad_cols(Th4, 64), _PERM),
                 jnp.zeros((64,), jnp.float32), False, 64)
  e5 = _segsum(p4, node2, he1, nch_z, 0)
  e5p = _tc_escale([e5], bg2, 32, 2)
  m5 = _segsum(e5p, he2, node1, nch_z, 0)

  wsp = jnp.zeros((64, 128), jnp.float32).at[:c_cls, :c_cls].set(Wlp[:c_cls])
  whp = jnp.zeros((64, 128), jnp.float32).at[:c_cls, :c_cls].set(Wlp[c_cls:])
  blpp = jnp.zeros((1, 128), jnp.float32).at[0, :c_cls].set(blp)
  bh4p = _pad_cols(bh4.reshape(1, -1), 64)
  out128 = _tc_final(a3, cnt2, rs, m5, dg2, bh4p, wsp, whp, blpp, c_cls)
  return out128[:, :c_cls]
